# Initial kernel scaffold; baseline (speedup 1.0000x reference)
import jax
import jax.numpy as jnp
from jax.experimental import pallas as pl


def kernel(x, edge_index, neighbor_ids, neighbor_w, emb, pos, W1, b1, q1, W2, b2, W3, b3, q2, W4, W5, b5, a_vec):
    raise NotImplementedError("write your pallas kernel here")


# speedup vs baseline: 1.1909x; 1.1909x over previous
import functools
import jax
import jax.numpy as jnp
from jax import lax
from jax.experimental import pallas as pl
from jax.experimental.pallas import tpu as pltpu

N_NODE = 100000
D = 100
B = 128
L = 20
N = B * L
K = 12
SB = 16
RB = SB * L
GRID_BC = B // SB
VB = 2048
GRID_D = (N_NODE + VB - 1) // VB


def _leaky(v, s):
    return jnp.where(v >= 0, v, s * v)


def _global_body(x_ref, nw_ref, nb_ref, hid_ref, w1_ref, b1_ref, q1_ref,
                 w2_ref, b2_ref, out_ref):
    xb = x_ref[...].astype(jnp.float32)
    s_mean = jnp.mean(xb, axis=1)
    w_soft = jax.nn.softmax(nw_ref[...], axis=-1)
    nb = nb_ref[...].reshape(SB, L, K, D)
    sh = s_mean[:, None, None, None] * nb
    feat = jnp.concatenate([sh, w_soft[..., None]], axis=-1)
    a = jnp.dot(feat.reshape(SB * L * K, D + 1), w1_ref[...],
                preferred_element_type=jnp.float32) + b1_ref[...]
    a = _leaky(a, 0.01)
    a = jnp.dot(a, q1_ref[...], preferred_element_type=jnp.float32)
    a = a.reshape(SB, L, K)
    alpha = jax.nn.softmax(a, axis=-1)
    h_n = jnp.sum(alpha[..., None] * nb, axis=2)
    hcat = jnp.concatenate([hid_ref[...], h_n.reshape(RB, D)], axis=1)
    hg = jnp.dot(hcat, w2_ref[...], preferred_element_type=jnp.float32)
    out_ref[...] = jnp.maximum(hg + b2_ref[...], 0.0)


def _run_global(x2d, neighbor_w, nbflat, hidden, W1, b1, q1, W2, b2):
    full = lambda shp: pl.BlockSpec(shp, lambda i: (0,) * len(shp))
    return pl.pallas_call(
        _global_body,
        grid=(GRID_BC,),
        in_specs=[
            pl.BlockSpec((SB, L), lambda i: (i, 0)),
            pl.BlockSpec((SB, L, K), lambda i: (i, 0, 0)),
            pl.BlockSpec((RB * K, D), lambda i: (i, 0)),
            pl.BlockSpec((RB, D), lambda i: (i, 0)),
            full((D + 1, D + 1)),
            full((1, D + 1)),
            full((D + 1, 1)),
            full((2 * D, D)),
            full((1, D)),
        ],
        out_specs=pl.BlockSpec((RB, D), lambda i: (i, 0)),
        out_shape=jax.ShapeDtypeStruct((N, D), jnp.float32),
    )(x2d, neighbor_w, nbflat, hidden, W1, b1.reshape(1, D + 1),
      q1.reshape(D + 1, 1), W2, b2.reshape(1, D))


def _local_body(hid_ref, hblk_ref, hg_ref, mask_ref, av_ref, pos_ref,
                w3_ref, b3_ref, w4_ref, w5_ref, b5_ref, q2_ref, out_ref):
    hid = hid_ref[...]
    hblk = hblk_ref[...]
    q = hblk * av_ref[...]
    pre = lax.dot_general(q, hid, (((1,), (1,)), ((), ())),
                          preferred_element_type=jnp.float32)
    e = _leaky(pre, 0.2)
    m = mask_ref[...] > 0.0
    e = jnp.where(m, e, -1e30)
    emax = jnp.max(e, axis=1, keepdims=True)
    ex = jnp.exp(e - emax)
    ex = jnp.where(m, ex, 0.0)
    att = ex / jnp.sum(ex, axis=1, keepdims=True)
    h = jnp.dot(att, hid, preferred_element_type=jnp.float32) + hg_ref[...]
    pos_rep = jnp.broadcast_to(pos_ref[...][:, None, :], (SB, L, D))
    pos_rep = pos_rep.reshape(RB, D)
    z = jnp.tanh(
        jnp.dot(jnp.concatenate([h, pos_rep], axis=1), w3_ref[...],
                preferred_element_type=jnp.float32) + b3_ref[...])
    s_sess = jnp.mean(h.reshape(SB, L, D), axis=1)
    s_rep = jnp.broadcast_to(s_sess[:, None, :], (SB, L, D)).reshape(RB, D)
    gate = jax.nn.sigmoid(
        jnp.dot(z, w4_ref[...], preferred_element_type=jnp.float32)
        + jnp.dot(s_rep, w5_ref[...], preferred_element_type=jnp.float32)
        + b5_ref[...])
    beta = jnp.dot(gate, q2_ref[...], preferred_element_type=jnp.float32)
    out_ref[...] = jnp.sum((beta * h).reshape(SB, L, D), axis=1)


def _run_local(hidden, h_global, mask, a_vec, pos, W3, b3, W4, W5, b5, q2):
    full = lambda shp: pl.BlockSpec(shp, lambda i: (0,) * len(shp))
    return pl.pallas_call(
        _local_body,
        grid=(GRID_BC,),
        in_specs=[
            full((N, D)),
            pl.BlockSpec((RB, D), lambda i: (i, 0)),
            pl.BlockSpec((RB, D), lambda i: (i, 0)),
            pl.BlockSpec((RB, N), lambda i: (i, 0)),
            full((1, D)),
            pl.BlockSpec((SB, D), lambda i: (i, 0)),
            full((2 * D, D)),
            full((1, D)),
            full((D, D)),
            full((D, D)),
            full((1, D)),
            full((D, 1)),
        ],
        out_specs=pl.BlockSpec((SB, D), lambda i: (i, 0)),
        out_shape=jax.ShapeDtypeStruct((B, D), jnp.float32),
    )(hidden, hidden, h_global, mask, a_vec.reshape(1, D), pos[:B],
      W3, b3.reshape(1, D), W4, W5, b5.reshape(1, D), q2.reshape(D, 1))


def _scores_body(s_ref, emb_ref, out_ref):
    out_ref[...] = lax.dot_general(
        s_ref[...], emb_ref[...], (((1,), (1,)), ((), ())),
        preferred_element_type=jnp.float32)


def _run_scores(S, emb):
    return pl.pallas_call(
        _scores_body,
        grid=(GRID_D,),
        in_specs=[
            pl.BlockSpec((B, D), lambda i: (0, 0)),
            pl.BlockSpec((VB, D), lambda i: (i, 0)),
        ],
        out_specs=pl.BlockSpec((B, VB), lambda i: (0, i)),
        out_shape=jax.ShapeDtypeStruct((B, N_NODE), jnp.float32),
    )(S, emb)


def kernel(x, edge_index, neighbor_ids, neighbor_w, emb, pos, W1, b1, q1,
           W2, b2, W3, b3, q2, W4, W5, b5, a_vec):
    hidden = emb[x]
    nbflat = emb[neighbor_ids.reshape(-1)]
    mask = (jnp.zeros((N, N), jnp.float32)
            .at[edge_index[0], edge_index[1]].set(1.0)
            .at[edge_index[1], edge_index[0]].set(1.0))

    h_global = _run_global(x.reshape(B, L), neighbor_w, nbflat, hidden,
                           W1, b1, q1, W2, b2)
    S = _run_local(hidden, h_global, mask, a_vec, pos, W3, b3, W4, W5, b5, q2)
    return _run_scores(S, emb)
